# Initial kernel scaffold; baseline (speedup 1.0000x reference)
import jax
import jax.numpy as jnp
from jax.experimental import pallas as pl


def kernel(x, edge_index, batch, W_rel, b_rel, W_root):
    raise NotImplementedError("write your pallas kernel here")


# speedup vs baseline: 32.8519x; 32.8519x over previous
import functools

import jax
import jax.numpy as jnp
from jax import lax
from jax.experimental import pallas as pl
from jax.experimental.pallas import tpu as pltpu
from jax.experimental.pallas import tpu_sc as plsc

HIDDEN = 128
N_NODES = 10000
N_EDGES = 320000
N_GRAPHS = 16

NC = 2
NS = 16
NW = NC * NS
EDGES_PER_W = N_EDGES // NW
LANES = 16
N_VREGS = N_NODES // LANES
E_VREGS = EDGES_PER_W // LANES


def _lin_body(x_ref, w_ref, b_ref, out_ref):
    out_ref[...] = (
        jnp.dot(x_ref[...], w_ref[...], preferred_element_type=jnp.float32)
        + b_ref[...]
    )


def _linear(x, w2, bias):
    return pl.pallas_call(
        _lin_body,
        out_shape=jax.ShapeDtypeStruct((N_NODES, 2), jnp.float32),
    )(x, w2, bias)


def _sc_body(y2_hbm, src_hbm, dst_hbm, out_hbm, y2_v, idx_v, vals_v, acc_sh):
    sc = lax.axis_index("c")
    sub = lax.axis_index("s")
    wid = sc * NS + sub

    pltpu.sync_copy(y2_hbm, y2_v)
    pltpu.sync_copy(src_hbm.at[pl.ds(wid * EDGES_PER_W, EDGES_PER_W)], idx_v)

    iota16 = lax.iota(jnp.int32, LANES)

    @pl.when(sub == 0)
    def _seed():
        def seed_body(j, _):
            n16 = (j * LANES + iota16) * 2 + 1
            z = plsc.load_gather(y2_v, [n16]) * 0.5
            vals_v[pl.ds(j * LANES, LANES)] = z
            return 0

        lax.fori_loop(0, N_VREGS, seed_body, 0)
        pltpu.sync_copy(vals_v, acc_sh)

    plsc.subcore_barrier()

    def gather_body(j, _):
        s16 = idx_v[pl.ds(j * LANES, LANES)]
        vals_v[pl.ds(j * LANES, LANES)] = plsc.load_gather(y2_v, [s16 * 2])
        return 0

    lax.fori_loop(0, E_VREGS, gather_body, 0)

    pltpu.sync_copy(dst_hbm.at[pl.ds(wid * EDGES_PER_W, EDGES_PER_W)], idx_v)
    pltpu.sync_copy(vals_v, acc_sh.at[idx_v], add=True)

    plsc.subcore_barrier()

    @pl.when(sub == 0)
    def _flush():
        pltpu.sync_copy(acc_sh, vals_v)
        pltpu.sync_copy(vals_v, out_hbm.at[sc])


def _sc_aggregate(y2, src, dst):
    mesh = plsc.VectorSubcoreMesh(core_axis_name="c", subcore_axis_name="s")
    kfn = pl.kernel(
        _sc_body,
        mesh=mesh,
        compiler_params=pltpu.CompilerParams(needs_layout_passes=False),
        out_type=jax.ShapeDtypeStruct((NC, N_NODES), jnp.float32),
        scratch_types=[
            pltpu.VMEM((2 * N_NODES,), jnp.float32),
            pltpu.VMEM((EDGES_PER_W,), jnp.int32),
            pltpu.VMEM((EDGES_PER_W,), jnp.float32),
            pltpu.VMEM_SHARED((N_NODES,), jnp.float32),
        ],
    )
    return kfn(y2.reshape(-1), src, dst)


def _pool_body(parts_ref, batch_ref, x_ref, out_ref):
    x_conv = parts_ref[0:1, :] + parts_ref[1:2, :]
    batch_b = jnp.broadcast_to(batch_ref[...], (N_GRAPHS, N_NODES))
    gids = lax.broadcasted_iota(jnp.int32, (N_GRAPHS, N_NODES), 0)
    mask = batch_b == gids

    xb = jnp.broadcast_to(x_conv, (N_GRAPHS, N_NODES))
    neg_inf = jnp.float32(-jnp.inf)
    seg_max = jnp.max(jnp.where(mask, xb, neg_inf), axis=1, keepdims=True)
    seg_max = jnp.where(seg_max > neg_inf, seg_max, 0.0)

    mx_node = jnp.sum(
        jnp.where(mask, jnp.broadcast_to(seg_max, (N_GRAPHS, N_NODES)), 0.0),
        axis=0, keepdims=True)
    ex = jnp.exp(x_conv - mx_node)
    exb = jnp.broadcast_to(ex, (N_GRAPHS, N_NODES))
    denom = jnp.sum(jnp.where(mask, exb, 0.0), axis=1, keepdims=True)
    den_node = jnp.sum(
        jnp.where(mask, jnp.broadcast_to(denom, (N_GRAPHS, N_NODES)), 0.0),
        axis=0, keepdims=True)
    scores = ex / (den_node + 1e-16)

    s_mat = jnp.where(mask, jnp.broadcast_to(scores, (N_GRAPHS, N_NODES)), 0.0)
    out_ref[...] = jnp.dot(s_mat, x_ref[...], preferred_element_type=jnp.float32)


def _pool(parts, batch_row, x):
    return pl.pallas_call(
        _pool_body,
        out_shape=jax.ShapeDtypeStruct((N_GRAPHS, HIDDEN), jnp.float32),
    )(parts, batch_row, x)


@jax.jit
def kernel(x, edge_index, batch, W_rel, b_rel, W_root):
    x = x.astype(jnp.float32)
    w2 = jnp.concatenate([W_rel, W_root], axis=1).astype(jnp.float32)
    bias = jnp.concatenate([jnp.zeros((1,), jnp.float32),
                            b_rel.astype(jnp.float32)]).reshape(1, 2)
    src = edge_index[0].astype(jnp.int32)
    dst = edge_index[1].astype(jnp.int32)
    batch_row = batch.astype(jnp.int32).reshape(1, N_NODES)

    y2 = _linear(x, w2, bias)
    parts = _sc_aggregate(y2, src, dst)
    return _pool(parts, batch_row, x)


# speedup vs baseline: 34.6731x; 1.0554x over previous
import functools

import jax
import jax.numpy as jnp
from jax import lax
from jax.experimental import pallas as pl
from jax.experimental.pallas import tpu as pltpu
from jax.experimental.pallas import tpu_sc as plsc

HIDDEN = 128
N_NODES = 10000
N_EDGES = 320000
N_GRAPHS = 16

NC = 2
NS = 16
NW = NC * NS
EDGES_PER_W = N_EDGES // NW
LANES = 16
N_VREGS = N_NODES // LANES
E_VREGS = EDGES_PER_W // LANES


def _lin_body(x_ref, wrel_ref, wroot_ref, b_ref, out_ref):
    wrel_q = wrel_ref[...].astype(jnp.bfloat16).astype(jnp.float32)
    y_rel = jnp.dot(x_ref[...], wrel_q, preferred_element_type=jnp.float32,
                    precision=lax.Precision.HIGHEST)
    z_root = jnp.dot(x_ref[...], wroot_ref[...], preferred_element_type=jnp.float32)
    out_ref[...] = jnp.concatenate([y_rel, z_root + b_ref[...]], axis=1)


def _linear(x, w_rel, w_root, b_rel):
    return pl.pallas_call(
        _lin_body,
        out_shape=jax.ShapeDtypeStruct((N_NODES, 2), jnp.float32),
    )(x, w_rel, w_root, b_rel)


def _sc_body(y2_hbm, ei_hbm, out_hbm, y2_v, src_v, dst_v, e_v, sem):
    sc = lax.axis_index("c")
    sub = lax.axis_index("s")
    wid = sc * NS + sub
    base = wid * EDGES_PER_W

    c_y2 = pltpu.async_copy(y2_hbm, y2_v, sem)
    c_src = pltpu.async_copy(ei_hbm.at[pl.ds(base, EDGES_PER_W)], src_v, sem)
    c_dst = pltpu.async_copy(
        ei_hbm.at[pl.ds(N_EDGES + base, EDGES_PER_W)], dst_v, sem)
    c_y2.wait()

    iota16 = lax.iota(jnp.int32, LANES)

    def seed_body(j, _):
        n16 = (j * LANES + iota16) * 2 + 1
        e_v[pl.ds(j * LANES, LANES)] = plsc.load_gather(y2_v, [n16]) * (1.0 / NW)
        return 0

    lax.fori_loop(0, N_VREGS, seed_body, 0)

    c_src.wait()
    c_dst.wait()

    def edge_body(j, _):
        s16 = src_v[pl.ds(j * LANES, LANES)]
        d16 = dst_v[pl.ds(j * LANES, LANES)]
        vals = plsc.load_gather(y2_v, [s16 * 2])
        plsc.addupdate_scatter(e_v, [d16], vals)
        return 0

    lax.fori_loop(0, E_VREGS, edge_body, 0)

    pltpu.sync_copy(e_v, out_hbm.at[wid])


def _sc_aggregate(y2, edge_index):
    mesh = plsc.VectorSubcoreMesh(core_axis_name="c", subcore_axis_name="s")
    kfn = pl.kernel(
        _sc_body,
        mesh=mesh,
        compiler_params=pltpu.CompilerParams(needs_layout_passes=False),
        out_type=jax.ShapeDtypeStruct((NW, N_NODES), jnp.float32),
        scratch_types=[
            pltpu.VMEM((2 * N_NODES,), jnp.float32),
            pltpu.VMEM((EDGES_PER_W,), jnp.int32),
            pltpu.VMEM((EDGES_PER_W,), jnp.int32),
            pltpu.VMEM((N_NODES,), jnp.float32),
            pltpu.SemaphoreType.DMA,
        ],
    )
    return kfn(y2.reshape(-1), edge_index.reshape(-1))


def _pool_body(parts_ref, batch_ref, x_ref, out_ref):
    x_conv = jnp.sum(parts_ref[...], axis=0, keepdims=True)
    batch_b = jnp.broadcast_to(batch_ref[...], (N_GRAPHS, N_NODES))
    gids = lax.broadcasted_iota(jnp.int32, (N_GRAPHS, N_NODES), 0)
    mask = batch_b == gids

    xb = jnp.broadcast_to(x_conv, (N_GRAPHS, N_NODES))
    neg_inf = jnp.float32(-jnp.inf)
    seg_max = jnp.max(jnp.where(mask, xb, neg_inf), axis=1, keepdims=True)
    seg_max = jnp.where(seg_max > neg_inf, seg_max, 0.0)

    mx_node = jnp.sum(
        jnp.where(mask, jnp.broadcast_to(seg_max, (N_GRAPHS, N_NODES)), 0.0),
        axis=0, keepdims=True)
    ex = jnp.exp(x_conv - mx_node)
    exb = jnp.broadcast_to(ex, (N_GRAPHS, N_NODES))
    denom = jnp.sum(jnp.where(mask, exb, 0.0), axis=1, keepdims=True)
    den_node = jnp.sum(
        jnp.where(mask, jnp.broadcast_to(denom, (N_GRAPHS, N_NODES)), 0.0),
        axis=0, keepdims=True)
    scores = ex / (den_node + 1e-16)

    s_mat = jnp.where(mask, jnp.broadcast_to(scores, (N_GRAPHS, N_NODES)), 0.0)
    out_ref[...] = jnp.dot(s_mat, x_ref[...], preferred_element_type=jnp.float32,
                           precision=lax.Precision.HIGHEST)


def _pool(parts, batch_row, x):
    return pl.pallas_call(
        _pool_body,
        out_shape=jax.ShapeDtypeStruct((N_GRAPHS, HIDDEN), jnp.float32),
    )(parts, batch_row, x)


@jax.jit
def kernel(x, edge_index, batch, W_rel, b_rel, W_root):
    x = x.astype(jnp.float32)
    ei = edge_index.astype(jnp.int32)
    batch_row = batch.astype(jnp.int32).reshape(1, N_NODES)

    y2 = _linear(x, W_rel.astype(jnp.float32), W_root.astype(jnp.float32),
                 b_rel.astype(jnp.float32).reshape(1, 1))
    parts = _sc_aggregate(y2, ei)
    return _pool(parts, batch_row, x)


# speedup vs baseline: 38.5999x; 1.1133x over previous
import functools

import jax
import jax.numpy as jnp
from jax import lax
from jax.experimental import pallas as pl
from jax.experimental.pallas import tpu as pltpu
from jax.experimental.pallas import tpu_sc as plsc

HIDDEN = 128
N_NODES = 10000
N_EDGES = 320000
N_GRAPHS = 16

NC = 2
NS = 16
NW = NC * NS
EDGES_PER_W = N_EDGES // NW
LANES = 16
E_VREGS = EDGES_PER_W // LANES

GRID = 16
ROWS = N_NODES // GRID

_NT = (((1,), (1,)), ((), ()))


def _lin_body(wrel_ref, wroot_ref, b_ref, x_ref, yrel_ref, z2_ref):
    x_blk = x_ref[...]
    x_hi = x_blk.astype(jnp.bfloat16)
    x_lo = (x_blk - x_hi.astype(jnp.float32)).astype(jnp.bfloat16)
    wrel_q = wrel_ref[...].astype(jnp.bfloat16)
    wroot_q = wroot_ref[...].astype(jnp.bfloat16)
    y = (lax.dot_general(wrel_q, x_hi, _NT, preferred_element_type=jnp.float32)
         + lax.dot_general(wrel_q, x_lo, _NT, preferred_element_type=jnp.float32))
    z = lax.dot_general(wroot_q, x_hi, _NT, preferred_element_type=jnp.float32)
    yrel_ref[...] = y.reshape(N_NODES)
    z2_ref[...] = (z + b_ref[...]).reshape(N_NODES)


def _linear(x, w_relT, w_rootT, b_rel):
    return pl.pallas_call(
        _lin_body,
        out_shape=[
            jax.ShapeDtypeStruct((N_NODES,), jnp.float32),
            jax.ShapeDtypeStruct((N_NODES,), jnp.float32),
        ],
    )(w_relT, w_rootT, b_rel, x)


def _sc_body(y_hbm, src_hbm, dst_hbm, zero_hbm, out_hbm,
             y_v, src_v, dst_v, e_v, sem):
    sc = lax.axis_index("c")
    sub = lax.axis_index("s")
    wid = sc * NS + sub
    base = wid * EDGES_PER_W

    c_y = pltpu.async_copy(y_hbm, y_v, sem)
    c_src = pltpu.async_copy(src_hbm.at[pl.ds(base, EDGES_PER_W)], src_v, sem)
    c_dst = pltpu.async_copy(dst_hbm.at[pl.ds(base, EDGES_PER_W)], dst_v, sem)
    c_zero = pltpu.async_copy(zero_hbm, e_v, sem)
    c_y.wait()
    c_src.wait()
    c_dst.wait()
    c_zero.wait()

    def edge_body(j):
        s16 = src_v[pl.ds(j * LANES, LANES)]
        d16 = dst_v[pl.ds(j * LANES, LANES)]
        vals = plsc.load_gather(y_v, [s16])
        plsc.addupdate_scatter(e_v, [d16], vals)

    plsc.parallel_loop(0, E_VREGS, 1, unroll=8)(edge_body)

    pltpu.sync_copy(e_v, out_hbm.at[wid])


def _sc_aggregate(y_rel, src, dst, zero):
    mesh = plsc.VectorSubcoreMesh(core_axis_name="c", subcore_axis_name="s")
    kfn = pl.kernel(
        _sc_body,
        mesh=mesh,
        compiler_params=pltpu.CompilerParams(needs_layout_passes=False),
        out_type=jax.ShapeDtypeStruct((NW, N_NODES), jnp.float32),
        scratch_types=[
            pltpu.VMEM((N_NODES,), jnp.float32),
            pltpu.VMEM((EDGES_PER_W,), jnp.int32),
            pltpu.VMEM((EDGES_PER_W,), jnp.int32),
            pltpu.VMEM((N_NODES,), jnp.float32),
            pltpu.SemaphoreType.DMA,
        ],
    )
    return kfn(y_rel, src, dst, zero)


def _scores_body(parts_ref, z2_ref, batch_ref, out_ref):
    x_conv = (jnp.sum(parts_ref[...], axis=0, keepdims=True)
              + z2_ref[...].reshape(1, N_NODES))
    batch_b = jnp.broadcast_to(batch_ref[...].reshape(1, N_NODES),
                               (N_GRAPHS, N_NODES))
    gids = lax.broadcasted_iota(jnp.int32, (N_GRAPHS, N_NODES), 0)
    mask = batch_b == gids

    xb = jnp.broadcast_to(x_conv, (N_GRAPHS, N_NODES))
    neg_inf = jnp.float32(-jnp.inf)
    seg_max = jnp.max(jnp.where(mask, xb, neg_inf), axis=1, keepdims=True)
    seg_max = jnp.where(seg_max > neg_inf, seg_max, 0.0)

    mx_node = jnp.sum(
        jnp.where(mask, jnp.broadcast_to(seg_max, (N_GRAPHS, N_NODES)), 0.0),
        axis=0, keepdims=True)
    ex = jnp.exp(x_conv - mx_node)
    exb = jnp.broadcast_to(ex, (N_GRAPHS, N_NODES))
    denom = jnp.sum(jnp.where(mask, exb, 0.0), axis=1, keepdims=True)
    den_node = jnp.sum(
        jnp.where(mask, jnp.broadcast_to(denom, (N_GRAPHS, N_NODES)), 0.0),
        axis=0, keepdims=True)
    out_ref[...] = (ex / (den_node + 1e-16)).reshape(N_NODES)


def _scores(parts, z2, batch):
    return pl.pallas_call(
        _scores_body,
        out_shape=jax.ShapeDtypeStruct((N_NODES,), jnp.float32),
    )(parts, z2, batch)


def _pool_body(scores_ref, batch_ref, x_ref, out_ref):
    batch_b = jnp.broadcast_to(batch_ref[...].reshape(1, N_NODES),
                               (N_GRAPHS, N_NODES))
    gids = lax.broadcasted_iota(jnp.int32, (N_GRAPHS, N_NODES), 0)
    s_row = jnp.broadcast_to(scores_ref[...].reshape(1, N_NODES),
                             (N_GRAPHS, N_NODES))
    s_mat = jnp.where(batch_b == gids, s_row, 0.0)
    out_ref[...] = jnp.dot(s_mat, x_ref[...],
                           preferred_element_type=jnp.float32,
                           precision=lax.Precision.HIGHEST)


def _pool(scores, batch, x):
    return pl.pallas_call(
        _pool_body,
        out_shape=jax.ShapeDtypeStruct((N_GRAPHS, HIDDEN), jnp.float32),
    )(scores, batch, x)


@jax.jit
def kernel(x, edge_index, batch, W_rel, b_rel, W_root):
    x = x.astype(jnp.float32)
    ei = edge_index.astype(jnp.int32)
    src = ei[0]
    dst = ei[1]
    batch_i = batch.astype(jnp.int32)
    zero = jnp.zeros((N_NODES,), jnp.float32)

    y_rel, z2 = _linear(x, W_rel.astype(jnp.float32).reshape(1, HIDDEN),
                        W_root.astype(jnp.float32).reshape(1, HIDDEN),
                        b_rel.astype(jnp.float32).reshape(1, 1))
    parts = _sc_aggregate(y_rel, src, dst, zero)
    scores = _scores(parts, z2, batch_i)
    return _pool(scores, batch_i, x)


# speedup vs baseline: 53.6338x; 1.3895x over previous
import functools

import jax
import jax.numpy as jnp
from jax import lax
from jax.experimental import pallas as pl
from jax.experimental.pallas import tpu as pltpu
from jax.experimental.pallas import tpu_sc as plsc

HIDDEN = 128
N_NODES = 10000
N_EDGES = 320000
N_GRAPHS = 16

NC = 2
NS = 16
NW = NC * NS
EDGES_PER_W = N_EDGES // NW
LANES = 16
E_VREGS = EDGES_PER_W // LANES

GRID = 16
ROWS = N_NODES // GRID

_NT = (((1,), (1,)), ((), ()))


def _lin_body(wrel_ref, wroot_ref, b_ref, x_ref, yrel_ref, z2_ref):
    x_blk = x_ref[...]
    x_hi = x_blk.astype(jnp.bfloat16)
    x_lo = (x_blk - x_hi.astype(jnp.float32)).astype(jnp.bfloat16)
    wrel_q = wrel_ref[...].astype(jnp.bfloat16)
    wroot_q = wroot_ref[...].astype(jnp.bfloat16)
    y = (lax.dot_general(wrel_q, x_hi, _NT, preferred_element_type=jnp.float32)
         + lax.dot_general(wrel_q, x_lo, _NT, preferred_element_type=jnp.float32))
    z = lax.dot_general(wroot_q, x_hi, _NT, preferred_element_type=jnp.float32)
    yrel_ref[...] = y.reshape(N_NODES)
    z2_ref[...] = (z + b_ref[...]).reshape(N_NODES)


def _linear(x, w_relT, w_rootT, b_rel):
    return pl.pallas_call(
        _lin_body,
        out_shape=[
            jax.ShapeDtypeStruct((N_NODES,), jnp.float32),
            jax.ShapeDtypeStruct((N_NODES,), jnp.float32),
        ],
    )(w_relT, w_rootT, b_rel, x)


E_SPAN = EDGES_PER_W + 112


def _sc_body(y_hbm, ei_hbm, zero_hbm, out_hbm, y_v, ei_v, e_v, sem):
    sc = lax.axis_index("c")
    sub = lax.axis_index("s")
    wid = sc * NS + sub
    base = wid * EDGES_PER_W
    aligned = (base // 128) * 128
    off = base - aligned

    c_y = pltpu.async_copy(y_hbm, y_v, sem)
    c_ei = pltpu.async_copy(ei_hbm.at[:, pl.ds(aligned, E_SPAN)], ei_v, sem)
    c_zero = pltpu.async_copy(zero_hbm, e_v, sem)
    c_y.wait()
    c_ei.wait()
    c_zero.wait()

    def edge_body(j):
        s16 = ei_v[0, pl.ds(off + j * LANES, LANES)]
        d16 = ei_v[1, pl.ds(off + j * LANES, LANES)]
        vals = plsc.load_gather(y_v, [s16])
        plsc.addupdate_scatter(e_v, [d16], vals)

    plsc.parallel_loop(0, E_VREGS, 1, unroll=8)(edge_body)

    pltpu.sync_copy(e_v, out_hbm.at[wid])


def _sc_aggregate(y_rel, ei, zero):
    mesh = plsc.VectorSubcoreMesh(core_axis_name="c", subcore_axis_name="s")
    kfn = pl.kernel(
        _sc_body,
        mesh=mesh,
        compiler_params=pltpu.CompilerParams(needs_layout_passes=False),
        out_type=jax.ShapeDtypeStruct((NW, N_NODES), jnp.float32),
        scratch_types=[
            pltpu.VMEM((N_NODES,), jnp.float32),
            pltpu.VMEM((2, E_SPAN), jnp.int32),
            pltpu.VMEM((N_NODES,), jnp.float32),
            pltpu.SemaphoreType.DMA,
        ],
    )
    return kfn(y_rel, ei, zero)


def _scores_body(parts_ref, z2_ref, batch_ref, out_ref):
    x_conv = (jnp.sum(parts_ref[...], axis=0, keepdims=True)
              + z2_ref[...].reshape(1, N_NODES))
    batch_b = jnp.broadcast_to(batch_ref[...].reshape(1, N_NODES),
                               (N_GRAPHS, N_NODES))
    gids = lax.broadcasted_iota(jnp.int32, (N_GRAPHS, N_NODES), 0)
    mask = batch_b == gids

    xb = jnp.broadcast_to(x_conv, (N_GRAPHS, N_NODES))
    neg_inf = jnp.float32(-jnp.inf)
    seg_max = jnp.max(jnp.where(mask, xb, neg_inf), axis=1, keepdims=True)
    seg_max = jnp.where(seg_max > neg_inf, seg_max, 0.0)

    mx_node = jnp.sum(
        jnp.where(mask, jnp.broadcast_to(seg_max, (N_GRAPHS, N_NODES)), 0.0),
        axis=0, keepdims=True)
    ex = jnp.exp(x_conv - mx_node)
    exb = jnp.broadcast_to(ex, (N_GRAPHS, N_NODES))
    denom = jnp.sum(jnp.where(mask, exb, 0.0), axis=1, keepdims=True)
    den_node = jnp.sum(
        jnp.where(mask, jnp.broadcast_to(denom, (N_GRAPHS, N_NODES)), 0.0),
        axis=0, keepdims=True)
    out_ref[...] = (ex / (den_node + 1e-16)).reshape(N_NODES)


def _scores(parts, z2, batch):
    return pl.pallas_call(
        _scores_body,
        out_shape=jax.ShapeDtypeStruct((N_NODES,), jnp.float32),
    )(parts, z2, batch)


def _pool_body(scores_ref, batch_ref, x_ref, out_ref):
    batch_b = jnp.broadcast_to(batch_ref[...].reshape(1, N_NODES),
                               (N_GRAPHS, N_NODES))
    gids = lax.broadcasted_iota(jnp.int32, (N_GRAPHS, N_NODES), 0)
    s_row = jnp.broadcast_to(scores_ref[...].reshape(1, N_NODES),
                             (N_GRAPHS, N_NODES))
    s_mat = jnp.where(batch_b == gids, s_row, 0.0)
    out_ref[...] = jnp.dot(s_mat, x_ref[...],
                           preferred_element_type=jnp.float32,
                           precision=lax.Precision.HIGHEST)


def _pool(scores, batch, x):
    return pl.pallas_call(
        _pool_body,
        out_shape=jax.ShapeDtypeStruct((N_GRAPHS, HIDDEN), jnp.float32),
    )(scores, batch, x)


@jax.jit
def kernel(x, edge_index, batch, W_rel, b_rel, W_root):
    x = x.astype(jnp.float32)
    ei = edge_index.astype(jnp.int32)
    batch_i = batch.astype(jnp.int32)
    zero = jnp.zeros((N_NODES,), jnp.float32)

    y_rel, z2 = _linear(x, W_rel.astype(jnp.float32).reshape(1, HIDDEN),
                        W_root.astype(jnp.float32).reshape(1, HIDDEN),
                        b_rel.astype(jnp.float32).reshape(1, 1))
    parts = _sc_aggregate(y_rel, ei, zero)
    scores = _scores(parts, z2, batch_i)
    return _pool(scores, batch_i, x)
